# baseline (speedup 1.0000x reference)
import jax
import jax.numpy as jnp
from jax import lax
from jax.experimental import pallas as pl
from jax.experimental.pallas import tpu as pltpu
from jax.experimental.pallas import tpu_sc as plsc

VOCAB = 1000000
D = 64
S = 200
B = 4096
NC = 2
NS = 16
NW = NC * NS
BPW = B // NW
L = 16
NACC = 4

assert S % NACC == 0


def _pool_body(ids_hbm, table_hbm, pooled_hbm, idx_v, idxt_v, acc_v,
               pooled_v, *sems):
    cid = lax.axis_index("c")
    sid = lax.axis_index("s")
    wid = sid * NC + cid
    base = wid * BPW

    pltpu.sync_copy(ids_hbm.at[pl.ds(base * S, BPW * S)], idx_v)

    row_base = [lax.iota(jnp.int32, L) * S + jnp.int32(g * L * S)
                for g in range(BPW // L)]

    def transpose_step(t, _):
        for g in range(BPW // L):
            vals = plsc.load_gather(idx_v, [row_base[g] + t])
            idxt_v[t, pl.ds(g * L, L)] = vals
        return 0

    lax.fori_loop(0, S, transpose_step, 0)

    def start(t, r, add):
        pltpu.async_copy(table_hbm.at[idxt_v.at[t]], acc_v.at[r], sems[r],
                         add=add)

    def wait_one(r):
        pltpu.make_async_copy(table_hbm.at[idxt_v.at[0]], acc_v.at[r],
                              sems[r]).wait()

    for r in range(NACC):
        start(r, r, False)

    def outer(i, _):
        for r in range(NACC):
            wait_one(r)
            start(i * NACC + r, r, True)
        return 0

    lax.fori_loop(1, S // NACC, outer, 0)
    for r in range(NACC):
        wait_one(r)

    inv = jnp.float32(1.0 / S)

    def combine(j, _):
        for c in range(4):
            v = acc_v[0, j, pl.ds(c * L, L)]
            for r in range(1, NACC):
                v = v + acc_v[r, j, pl.ds(c * L, L)]
            pooled_v[j, pl.ds(c * L, L)] = v * inv
        return 0

    lax.fori_loop(0, BPW, combine, 0)

    pltpu.sync_copy(pooled_v, pooled_hbm.at[pl.ds(base, BPW)])


@jax.jit
def _pool(ids_flat, table):
    mesh = plsc.VectorSubcoreMesh(core_axis_name="c", subcore_axis_name="s")
    return pl.kernel(
        _pool_body,
        out_type=jax.ShapeDtypeStruct((B, D), jnp.float32),
        mesh=mesh,
        scratch_types=[
            pltpu.VMEM((BPW * S,), jnp.int32),
            pltpu.VMEM((S, BPW), jnp.int32),
            pltpu.VMEM((NACC, BPW, D), jnp.float32),
            pltpu.VMEM((BPW, D), jnp.float32),
        ] + [pltpu.SemaphoreType.DMA] * NACC,
        compiler_params=pltpu.CompilerParams(use_tc_tiling_on_sc=False,
                                             needs_layout_passes=False),
    )(ids_flat, table)


def _mlp_body(x_ref, w1_ref, b1_ref, w2_ref, b2_ref, o_ref):
    x = x_ref[...]
    h = jnp.dot(x, w1_ref[...], preferred_element_type=jnp.float32)
    h = jnp.maximum(h + b1_ref[...], 0.0)
    o_ref[...] = (
        jnp.dot(h, w2_ref[...], preferred_element_type=jnp.float32)
        + b2_ref[...])


@jax.jit
def _mlp(pooled, W1, b1, W2, b2):
    return pl.pallas_call(
        _mlp_body,
        out_shape=jax.ShapeDtypeStruct((B, 2), jnp.float32),
    )(pooled, W1, b1.reshape(1, -1), W2, b2.reshape(1, -1))


def kernel(input_ids, emb_table, W1, b1, W2, b2):
    ids_flat = input_ids.astype(jnp.int32).reshape(-1)
    pooled = _pool(ids_flat, emb_table)
    return _mlp(pooled, W1, b1, W2, b2)


# speedup vs baseline: 1.0211x; 1.0211x over previous
import jax
import jax.numpy as jnp
from jax import lax
from jax.experimental import pallas as pl
from jax.experimental.pallas import tpu as pltpu
from jax.experimental.pallas import tpu_sc as plsc

VOCAB = 1000000
D = 64
S = 200
B = 4096
NC = 2
NS = 16
NW = NC * NS
BPW = B // NW
L = 16
NACC = 8

assert S % NACC == 0


def _pool_body(ids_hbm, table_hbm, pooled_hbm, idx_v, idxt_v, acc_v,
               pooled_v, *sems):
    cid = lax.axis_index("c")
    sid = lax.axis_index("s")
    wid = sid * NC + cid
    base = wid * BPW

    pltpu.sync_copy(ids_hbm.at[pl.ds(base, BPW)], idx_v)

    rows = [lax.iota(jnp.int32, L) + jnp.int32(g * L)
            for g in range(BPW // L)]

    def transpose_step(t, _):
        col = jnp.full((L,), t, jnp.int32)
        for g in range(BPW // L):
            idxt_v[t, pl.ds(g * L, L)] = plsc.load_gather(
                idx_v, [rows[g], col])
        return 0

    lax.fori_loop(0, S, transpose_step, 0)

    def start(t, r, add):
        pltpu.async_copy(table_hbm.at[idxt_v.at[t]], acc_v.at[r], sems[r],
                         add=add)

    def wait_one(r):
        pltpu.make_async_copy(table_hbm.at[idxt_v.at[0]], acc_v.at[r],
                              sems[r]).wait()

    for r in range(NACC):
        start(r, r, False)

    def outer(i, _):
        for r in range(NACC):
            wait_one(r)
            start(i * NACC + r, r, True)
        return 0

    lax.fori_loop(1, S // NACC, outer, 0)
    for r in range(NACC):
        wait_one(r)

    inv = jnp.float32(1.0 / S)

    def combine(j, _):
        for c in range(4):
            v = acc_v[0, j, pl.ds(c * L, L)]
            for r in range(1, NACC):
                v = v + acc_v[r, j, pl.ds(c * L, L)]
            pooled_v[j, pl.ds(c * L, L)] = v * inv
        return 0

    lax.fori_loop(0, BPW, combine, 0)

    pltpu.sync_copy(pooled_v, pooled_hbm.at[pl.ds(base, BPW)])


@jax.jit
def _pool(ids, table):
    mesh = plsc.VectorSubcoreMesh(core_axis_name="c", subcore_axis_name="s")
    return pl.kernel(
        _pool_body,
        out_type=jax.ShapeDtypeStruct((B, D), jnp.float32),
        mesh=mesh,
        scratch_types=[
            pltpu.VMEM((BPW, S), jnp.int32),
            pltpu.VMEM((S, BPW), jnp.int32),
            pltpu.VMEM((NACC, BPW, D), jnp.float32),
            pltpu.VMEM((BPW, D), jnp.float32),
        ] + [pltpu.SemaphoreType.DMA] * NACC,
        compiler_params=pltpu.CompilerParams(use_tc_tiling_on_sc=False,
                                             needs_layout_passes=False),
    )(ids, table)


def _mlp_body(x_ref, w1_ref, b1_ref, w2_ref, b2_ref, o_ref):
    x = x_ref[...]
    h = jnp.dot(x, w1_ref[...], preferred_element_type=jnp.float32)
    h = jnp.maximum(h + b1_ref[...], 0.0)
    o_ref[...] = (
        jnp.dot(h, w2_ref[...], preferred_element_type=jnp.float32)
        + b2_ref[...])


@jax.jit
def _mlp(pooled, W1, b1, W2, b2):
    return pl.pallas_call(
        _mlp_body,
        out_shape=jax.ShapeDtypeStruct((B, 2), jnp.float32),
    )(pooled, W1, b1.reshape(1, -1), W2, b2.reshape(1, -1))


def kernel(input_ids, emb_table, W1, b1, W2, b2):
    pooled = _pool(input_ids.astype(jnp.int32), emb_table)
    return _mlp(pooled, W1, b1, W2, b2)


# speedup vs baseline: 1.1082x; 1.0853x over previous
import jax
import jax.numpy as jnp
from jax import lax
from jax.experimental import pallas as pl
from jax.experimental.pallas import tpu as pltpu
from jax.experimental.pallas import tpu_sc as plsc

VOCAB = 1000000
D = 64
DP = 128
S = 200
B = 4096
NC = 2
NS = 16
NW = NC * NS
BPW = B // NW
L = 16
NACC = 4
TCOL = 2048

assert S % NACC == 0


def _pad_body(tt_ref, o_ref):
    x = tt_ref[...]
    y = x.T
    o_ref[...] = jnp.concatenate([y, jnp.zeros_like(y)], axis=1)


@jax.jit
def _pad(table_t):
    grid = (VOCAB + TCOL - 1) // TCOL
    return pl.pallas_call(
        _pad_body,
        grid=(grid,),
        in_specs=[pl.BlockSpec((D, TCOL), lambda i: (0, i))],
        out_specs=pl.BlockSpec((TCOL, DP), lambda i: (i, 0)),
        out_shape=jax.ShapeDtypeStruct((VOCAB, DP), jnp.float32),
    )(table_t)


def _pool_body(idst_hbm, table_hbm, pooled_hbm, idxt_v, acc_v, pooled_v,
               *sems):
    cid = lax.axis_index("c")
    sid = lax.axis_index("s")
    wid = sid * NC + cid
    base = wid * BPW

    pltpu.sync_copy(idst_hbm.at[:, pl.ds(base, BPW)], idxt_v)

    def start(t, r, add):
        pltpu.async_copy(table_hbm.at[idxt_v.at[t]], acc_v.at[r], sems[r],
                         add=add)

    def wait_one(r):
        pltpu.make_async_copy(table_hbm.at[idxt_v.at[0]], acc_v.at[r],
                              sems[r]).wait()

    for r in range(NACC):
        start(r, r, False)

    def outer(i, _):
        for r in range(NACC):
            wait_one(r)
            start(i * NACC + r, r, True)
        return 0

    lax.fori_loop(1, S // NACC, outer, 0)
    for r in range(NACC):
        wait_one(r)

    inv = jnp.float32(1.0 / S)

    def combine(j2, _):
        for half in range(2):
            j = j2 * 2 + half
            for c in range(4):
                v = acc_v[0, j, pl.ds(c * L, L)]
                for r in range(1, NACC):
                    v = v + acc_v[r, j, pl.ds(c * L, L)]
                pooled_v[j2, pl.ds(half * D + c * L, L)] = v * inv
        return 0

    lax.fori_loop(0, BPW // 2, combine, 0)

    pltpu.sync_copy(pooled_v, pooled_hbm.at[pl.ds(wid * (BPW // 2),
                                                  BPW // 2)])


@jax.jit
def _pool(ids_t, table_pad):
    mesh = plsc.VectorSubcoreMesh(core_axis_name="c", subcore_axis_name="s")
    return pl.kernel(
        _pool_body,
        out_type=jax.ShapeDtypeStruct((B // 2, DP), jnp.float32),
        mesh=mesh,
        scratch_types=[
            pltpu.VMEM((S, BPW), jnp.int32),
            pltpu.VMEM((NACC, BPW, DP), jnp.float32),
            pltpu.VMEM((BPW // 2, DP), jnp.float32),
        ] + [pltpu.SemaphoreType.DMA] * NACC,
        compiler_params=pltpu.CompilerParams(use_tc_tiling_on_sc=True,
                                             needs_layout_passes=False),
    )(ids_t, table_pad)


def _mlp_body(x_ref, w1_ref, b1_ref, w2_ref, b2_ref, o_ref):
    x = x_ref[...]
    h = jnp.dot(x, w1_ref[...], preferred_element_type=jnp.float32)
    h = jnp.maximum(h + b1_ref[...], 0.0)
    o_ref[...] = (
        jnp.dot(h, w2_ref[...], preferred_element_type=jnp.float32)
        + b2_ref[...])


@jax.jit
def _mlp(pooled, W1, b1, W2, b2):
    return pl.pallas_call(
        _mlp_body,
        out_shape=jax.ShapeDtypeStruct((B, 2), jnp.float32),
    )(pooled, W1, b1.reshape(1, -1), W2, b2.reshape(1, -1))


def kernel(input_ids, emb_table, W1, b1, W2, b2):
    table_pad = _pad(emb_table.T)
    ids_t = input_ids.astype(jnp.int32).T
    pooled2 = _pool(ids_t, table_pad)
    pooled = pooled2.reshape(B, D)
    return _mlp(pooled, W1, b1, W2, b2)


# speedup vs baseline: 1.3849x; 1.2497x over previous
import jax
import jax.numpy as jnp
from jax import lax
from jax.experimental import pallas as pl
from jax.experimental.pallas import tpu as pltpu
from jax.experimental.pallas import tpu_sc as plsc

VOCAB = 1000000
D = 64
DP = 128
S = 200
B = 4096
NC = 2
NS = 16
NW = NC * NS
BPW = B // NW
L = 16
NACC = 5
TCOL = 4096

assert S % NACC == 0


def _pad_body(tt_ref, o_ref):
    x = tt_ref[...]
    o_ref[:, pl.ds(0, D)] = x.T


@jax.jit
def _pad(table_t):
    grid = (VOCAB + TCOL - 1) // TCOL
    return pl.pallas_call(
        _pad_body,
        grid=(grid,),
        in_specs=[pl.BlockSpec((D, TCOL), lambda i: (0, i))],
        out_specs=pl.BlockSpec((TCOL, DP), lambda i: (i, 0)),
        out_shape=jax.ShapeDtypeStruct((VOCAB, DP), jnp.float32),
    )(table_t)


def _pool_body(idst_hbm, table_hbm, pooled_hbm, idxt_v, acc_v, pooled_v,
               *sems):
    cid = lax.axis_index("c")
    sid = lax.axis_index("s")
    wid = sid * NC + cid
    base = wid * BPW

    pltpu.sync_copy(idst_hbm.at[:, pl.ds(base, BPW)], idxt_v)

    def start(t, r, add):
        pltpu.async_copy(table_hbm.at[idxt_v.at[t]], acc_v.at[r], sems[r],
                         add=add)

    def wait_one(r):
        pltpu.make_async_copy(table_hbm.at[idxt_v.at[0]], acc_v.at[r],
                              sems[r]).wait()

    for r in range(NACC):
        start(r, r, False)

    def outer(i, _):
        for r in range(NACC):
            wait_one(r)
            start(i * NACC + r, r, True)
        return 0

    lax.fori_loop(1, S // NACC, outer, 0)
    for r in range(NACC):
        wait_one(r)

    inv = jnp.float32(1.0 / S)

    def combine(j2, _):
        for half in range(2):
            j = j2 * 2 + half
            for c in range(4):
                v = acc_v[0, j, pl.ds(c * L, L)]
                for r in range(1, NACC):
                    v = v + acc_v[r, j, pl.ds(c * L, L)]
                pooled_v[j2, pl.ds(half * D + c * L, L)] = v * inv
        return 0

    lax.fori_loop(0, BPW // 2, combine, 0)

    pltpu.sync_copy(pooled_v, pooled_hbm.at[pl.ds(wid * (BPW // 2),
                                                  BPW // 2)])


@jax.jit
def _pool(ids_t, table_pad):
    mesh = plsc.VectorSubcoreMesh(core_axis_name="c", subcore_axis_name="s")
    return pl.kernel(
        _pool_body,
        out_type=jax.ShapeDtypeStruct((B // 2, DP), jnp.float32),
        mesh=mesh,
        scratch_types=[
            pltpu.VMEM((S, BPW), jnp.int32),
            pltpu.VMEM((NACC, BPW, DP), jnp.float32),
            pltpu.VMEM((BPW // 2, DP), jnp.float32),
        ] + [pltpu.SemaphoreType.DMA] * NACC,
        compiler_params=pltpu.CompilerParams(use_tc_tiling_on_sc=True,
                                             needs_layout_passes=False),
    )(ids_t, table_pad)


def _mlp_body(x_ref, w1_ref, b1_ref, w2_ref, b2_ref, o_ref):
    x = x_ref[...]
    h = jnp.dot(x, w1_ref[...], preferred_element_type=jnp.float32)
    h = jnp.maximum(h + b1_ref[...], 0.0)
    o_ref[...] = (
        jnp.dot(h, w2_ref[...], preferred_element_type=jnp.float32)
        + b2_ref[...])


@jax.jit
def _mlp(pooled, W1, b1, W2, b2):
    return pl.pallas_call(
        _mlp_body,
        out_shape=jax.ShapeDtypeStruct((B, 2), jnp.float32),
    )(pooled, W1, b1.reshape(1, -1), W2, b2.reshape(1, -1))


def kernel(input_ids, emb_table, W1, b1, W2, b2):
    table_pad = _pad(emb_table.T)
    ids_t = input_ids.astype(jnp.int32).T
    pooled2 = _pool(ids_t, table_pad)
    pooled = pooled2.reshape(B, D)
    return _mlp(pooled, W1, b1, W2, b2)
